# baseline (speedup 1.0000x reference)
import functools

import jax
import jax.numpy as jnp
from jax import lax
from jax.experimental import pallas as pl
from jax.experimental.pallas import tpu as pltpu
from jax.experimental.pallas import tpu_sc as plsc


_K1_CB = 8192


def _relayout_body(tt_ref, out_ref):
    out_ref[...] = jnp.transpose(tt_ref[...])


def _relayout(table_t):
    E, V = table_t.shape
    grid = (V + _K1_CB - 1) // _K1_CB
    return pl.pallas_call(
        _relayout_body,
        grid=(grid,),
        in_specs=[pl.BlockSpec((E, _K1_CB), lambda i: (0, i))],
        out_specs=pl.BlockSpec((_K1_CB, E), lambda i: (i, 0)),
        out_shape=jax.ShapeDtypeStruct((V, E), jnp.float32),
    )(table_t)


def _make_sc_gather(V8, B, F, E):
    info = plsc.get_sparse_core_info()
    nc = info.num_cores
    nw = nc * info.num_subcores
    n = B * F // nw
    nb = n // F
    ch = 128
    nch = n // ch
    mesh = plsc.VectorSubcoreMesh(core_axis_name="c", subcore_axis_name="s")

    @functools.partial(
        pl.kernel,
        mesh=mesh,
        compiler_params=pltpu.CompilerParams(needs_layout_passes=False),
        out_type=jax.ShapeDtypeStruct((B, F * E), jnp.float32),
        scratch_types=[
            pltpu.VMEM((n,), jnp.int32),
            pltpu.VMEM((n,), jnp.int32),
            pltpu.VMEM((n,), jnp.int32),
            pltpu.VMEM((n,), jnp.int32),
            pltpu.VMEM((2, ch, 8 * E), jnp.float32),
            pltpu.VMEM((nb, F * E), jnp.float32),
            pltpu.SemaphoreType.DMA,
            pltpu.SemaphoreType.DMA,
        ],
    )
    def gather(tab_hbm, rows_hbm, sub_hbm, dstb_hbm, dstc_hbm, out_hbm,
               rows_v, sub_v, dstb_v, dstc_v, buf, out_v, sem0, sem1):
        wid = lax.axis_index("s") * nc + lax.axis_index("c")
        base = wid * n
        pltpu.sync_copy(rows_hbm.at[pl.ds(base, n)], rows_v)
        pltpu.sync_copy(sub_hbm.at[pl.ds(base, n)], sub_v)
        pltpu.sync_copy(dstb_hbm, dstb_v)
        pltpu.sync_copy(dstc_hbm, dstc_v)
        sems = [sem0, sem1]
        iota = lax.iota(jnp.int32, 16)

        def fire(g, b):
            pltpu.async_copy(
                tab_hbm.at[rows_v.at[pl.ds(g * ch, ch)]], buf.at[b], sems[b])

        def drain(b):
            pltpu.make_async_copy(
                tab_hbm.at[rows_v.at[pl.ds(0, ch)]], buf.at[b], sems[b]).wait()

        def extract(g, b):
            for q in range(ch // 16):
                s0 = g * ch + q * 16
                bat = dstb_v[pl.ds(s0, 16)]
                col0 = dstc_v[pl.ds(s0, 16)]
                sv = sub_v[pl.ds(s0, 16)]
                lrow = q * 16 + iota
                for m in range(E):
                    v = plsc.load_gather(buf.at[b], [lrow, sv + m])
                    plsc.store_scatter(out_v, [bat, col0 + m], v)

        fire(0, 0)

        def pair(p, carry):
            g0 = 2 * p
            fire(g0 + 1, 1)
            drain(0)
            extract(g0, 0)

            @pl.when(g0 + 2 < nch)
            def _():
                fire(g0 + 2, 0)

            drain(1)
            extract(g0 + 1, 1)
            return carry

        lax.fori_loop(0, nch // 2, pair, 0)
        pltpu.sync_copy(out_v, out_hbm.at[pl.ds(wid * nb, nb)])

    return gather


def _dense_body(fe_ref, lw_ref, t2_ref, w1_ref, b1_ref, g1_ref, be1_ref,
                w2_ref, b2_ref, g2_ref, be2_ref, wfc_ref, bfc_ref, out_ref,
                *, lin_dim):
    f32 = jnp.float32
    fe = fe_ref[...]
    lz = jnp.dot(fe, lw_ref[...], preferred_element_type=f32)
    lp = jnp.dot(fe * fe, t2_ref[...], preferred_element_type=f32)
    w1 = w1_ref[...]
    y = (jnp.dot(lz, w1[:lin_dim], preferred_element_type=f32)
         + jnp.dot(lp, w1[lin_dim:], preferred_element_type=f32)
         + b1_ref[...])

    def bn_relu(y, g, b):
        m = jnp.mean(y, axis=0, keepdims=True)
        c = y - m
        v = jnp.mean(c * c, axis=0, keepdims=True)
        return jnp.maximum(g * c * lax.rsqrt(v + 1e-5) + b, 0.0)

    y = bn_relu(y, g1_ref[...], be1_ref[...])
    y = jnp.dot(y, w2_ref[...], preferred_element_type=f32) + b2_ref[...]
    y = bn_relu(y, g2_ref[...], be2_ref[...])
    out_ref[...] = jnp.dot(y, wfc_ref[...], preferred_element_type=f32) + bfc_ref[...]


def _dense(fe, lw, t2, w1, b1, g1, be1, w2, b2, g2, be2, wfc, bfc, lin_dim):
    B = fe.shape[0]
    return pl.pallas_call(
        functools.partial(_dense_body, lin_dim=lin_dim),
        out_shape=jax.ShapeDtypeStruct((B, 1), jnp.float32),
    )(fe, lw, t2, w1, b1, g1, be1, w2, b2, g2, be2, wfc, bfc)


def kernel(feat_index, feat_value, emb_table, linear_weights, theta,
           W1, b1, g1, be1, W2, b2, g2, be2, Wfc, bfc):
    B, F = feat_index.shape
    V, E = emb_table.shape
    lin_dim = linear_weights.shape[0]

    table_rm = _relayout(jnp.transpose(emb_table))
    table2 = table_rm.reshape(V // 8, 8 * E)

    idx = feat_index.reshape(-1).astype(jnp.int32)
    rows = idx >> 3
    sub = (idx & 7) * E
    n = B * F // 32
    r = jnp.arange(n, dtype=jnp.int32)
    dstb = r // F
    dstc = (r % F) * E
    fe = _make_sc_gather(V // 8, B, F, E)(table2, rows, sub, dstb, dstc)

    lw = linear_weights.reshape(lin_dim, F * E).T
    t2 = jnp.repeat(theta * theta, E, axis=1).T

    return _dense(fe, lw, t2,
                  W1, b1.reshape(1, -1), g1.reshape(1, -1), be1.reshape(1, -1),
                  W2, b2.reshape(1, -1), g2.reshape(1, -1), be2.reshape(1, -1),
                  Wfc, bfc.reshape(1, -1), lin_dim)


# speedup vs baseline: 1.1517x; 1.1517x over previous
import functools

import jax
import jax.numpy as jnp
from jax import lax
from jax.experimental import pallas as pl
from jax.experimental.pallas import tpu as pltpu
from jax.experimental.pallas import tpu_sc as plsc


_K1_CB = 32768


def _relayout_body(tt_ref, out_ref):
    x = tt_ref[...]
    eye = jnp.eye(x.shape[0], dtype=x.dtype)
    out_ref[...] = lax.dot_general(
        x, eye, (((0,), (0,)), ((), ())),
        preferred_element_type=jnp.float32)


def _relayout(table_t):
    E, V = table_t.shape
    grid = (V + _K1_CB - 1) // _K1_CB
    return pl.pallas_call(
        _relayout_body,
        grid=(grid,),
        in_specs=[pl.BlockSpec((E, _K1_CB), lambda i: (0, i))],
        out_specs=pl.BlockSpec((_K1_CB, E), lambda i: (i, 0)),
        out_shape=jax.ShapeDtypeStruct((V, E), jnp.float32),
    )(table_t)


def _make_sc_gather(V, E, BF):
    info = plsc.get_sparse_core_info()
    nc = info.num_cores
    nw = nc * info.num_subcores
    n = BF // nw
    mesh = plsc.VectorSubcoreMesh(core_axis_name="c", subcore_axis_name="s")

    @functools.partial(
        pl.kernel,
        mesh=mesh,
        compiler_params=pltpu.CompilerParams(use_tc_tiling_on_sc=False),
        out_type=jax.ShapeDtypeStruct((BF, E), jnp.float32),
        scratch_types=[
            pltpu.VMEM((n,), jnp.int32),
            pltpu.VMEM((n, E), jnp.float32),
            pltpu.SemaphoreType.DMA,
        ],
    )
    def gather(tab_hbm, idx_hbm, out_hbm, idx_v, rows_v, sem):
        wid = lax.axis_index("s") * nc + lax.axis_index("c")
        base = wid * n
        pltpu.sync_copy(idx_hbm.at[pl.ds(base, n)], idx_v)
        pltpu.async_copy(tab_hbm.at[idx_v], rows_v, sem).wait()
        pltpu.sync_copy(rows_v, out_hbm.at[pl.ds(base, n)])

    return gather


def _dense_body(fe_ref, lw_ref, t2_ref, w1_ref, b1_ref, g1_ref, be1_ref,
                w2_ref, b2_ref, g2_ref, be2_ref, wfc_ref, bfc_ref, out_ref,
                *, lin_dim):
    f32 = jnp.float32
    fe = fe_ref[...]
    lz = jnp.dot(fe, lw_ref[...], preferred_element_type=f32)
    lp = jnp.dot(fe * fe, t2_ref[...], preferred_element_type=f32)
    w1 = w1_ref[...]
    y = (jnp.dot(lz, w1[:lin_dim], preferred_element_type=f32)
         + jnp.dot(lp, w1[lin_dim:], preferred_element_type=f32)
         + b1_ref[...])

    def bn_relu(y, g, b):
        m = jnp.mean(y, axis=0, keepdims=True)
        c = y - m
        v = jnp.mean(c * c, axis=0, keepdims=True)
        return jnp.maximum(g * c * lax.rsqrt(v + 1e-5) + b, 0.0)

    y = bn_relu(y, g1_ref[...], be1_ref[...])
    y = jnp.dot(y, w2_ref[...], preferred_element_type=f32) + b2_ref[...]
    y = bn_relu(y, g2_ref[...], be2_ref[...])
    out_ref[...] = jnp.dot(y, wfc_ref[...], preferred_element_type=f32) + bfc_ref[...]


def _dense(fe, lw, t2, w1, b1, g1, be1, w2, b2, g2, be2, wfc, bfc, lin_dim):
    B = fe.shape[0]
    return pl.pallas_call(
        functools.partial(_dense_body, lin_dim=lin_dim),
        out_shape=jax.ShapeDtypeStruct((B, 1), jnp.float32),
    )(fe, lw, t2, w1, b1, g1, be1, w2, b2, g2, be2, wfc, bfc)


def kernel(feat_index, feat_value, emb_table, linear_weights, theta,
           W1, b1, g1, be1, W2, b2, g2, be2, Wfc, bfc):
    B, F = feat_index.shape
    V, E = emb_table.shape
    lin_dim = linear_weights.shape[0]

    table_rm = _relayout(jnp.transpose(emb_table))

    idx = feat_index.reshape(-1).astype(jnp.int32)
    fe = _make_sc_gather(V, E, B * F)(table_rm, idx)
    fe = fe.reshape(B, F * E)

    lw = linear_weights.reshape(lin_dim, F * E).T
    t2 = jnp.repeat(theta * theta, E, axis=1).T

    return _dense(fe, lw, t2,
                  W1, b1.reshape(1, -1), g1.reshape(1, -1), be1.reshape(1, -1),
                  W2, b2.reshape(1, -1), g2.reshape(1, -1), be2.reshape(1, -1),
                  Wfc, bfc.reshape(1, -1), lin_dim)


# speedup vs baseline: 2.1745x; 1.8881x over previous
import functools

import jax
import jax.numpy as jnp
from jax import lax
from jax.experimental import pallas as pl
from jax.experimental.pallas import tpu as pltpu
from jax.experimental.pallas import tpu_sc as plsc


def _sc_info():
    info = plsc.get_sparse_core_info()
    return info.num_cores, info.num_cores * info.num_subcores


_CHUNK = 2048


def _make_sc_relayout(V, E):
    nc, nw = _sc_info()
    nfull = V // _CHUNK
    tail = V - nfull * _CHUNK
    per_w = (nfull + nw - 1) // nw
    vpad = ((V + 127) // 128) * 128
    mesh = plsc.VectorSubcoreMesh(core_axis_name="c", subcore_axis_name="s")

    @functools.partial(
        pl.kernel,
        mesh=mesh,
        compiler_params=pltpu.CompilerParams(needs_layout_passes=False),
        out_type=jax.ShapeDtypeStruct((vpad // 8, 8 * E), jnp.float32),
        scratch_types=[
            pltpu.VMEM((E, _CHUNK), jnp.float32),
            pltpu.VMEM((_CHUNK // 8, 8 * E), jnp.float32),
        ],
    )
    def relayout(tt_hbm, out_hbm, ib, ob):
        wid = lax.axis_index("s") * nc + lax.axis_index("c")
        iota = lax.iota(jnp.int32, 16)
        i8 = iota >> 3
        i16 = (iota & 7) * E
        zero = iota * 0
        idx1 = [i16 + j for j in range(E)]
        jv = [zero + j for j in range(E)]

        def do_chunk(col0, ncols):
            col0 = pl.multiple_of(col0, 128)
            pltpu.sync_copy(tt_hbm.at[:, pl.ds(col0, ncols)],
                            ib.at[:, pl.ds(0, ncols)])

            def q_body(q, carry):
                idx0 = 2 * q + i8
                cols = q * 16 + iota
                for j in range(E):
                    v = plsc.load_gather(ib, [jv[j], cols])
                    plsc.store_scatter(ob, [idx0, idx1[j]], v)
                return carry

            lax.fori_loop(0, ncols // 16, q_body, 0)
            pltpu.sync_copy(ob.at[pl.ds(0, ncols // 8)],
                            out_hbm.at[pl.ds(pl.multiple_of(col0 // 8, 8),
                                             ncols // 8)])

        def k_body(k, carry):
            t = wid * per_w + k

            @pl.when(t < nfull)
            def _():
                do_chunk(t * _CHUNK, _CHUNK)

            return carry

        lax.fori_loop(0, per_w, k_body, 0)

        if tail:
            t512 = (tail // 512) * 512

            @pl.when(wid == nw - 1)
            def _():
                if t512:
                    do_chunk(nfull * _CHUNK, t512)
                do_chunk(wid * 0 + nfull * _CHUNK + t512, 128)

    return relayout


def _make_sc_gather(B, F, E):
    nc, nw = _sc_info()
    n = B * F // nw
    nb = n // F
    ch = 128
    nch = n // ch
    mesh = plsc.VectorSubcoreMesh(core_axis_name="c", subcore_axis_name="s")

    @functools.partial(
        pl.kernel,
        mesh=mesh,
        compiler_params=pltpu.CompilerParams(needs_layout_passes=False),
        out_type=jax.ShapeDtypeStruct((B, F * E), jnp.float32),
        scratch_types=[
            pltpu.VMEM((n,), jnp.int32),
            pltpu.VMEM((n,), jnp.int32),
            pltpu.VMEM((n,), jnp.int32),
            pltpu.VMEM((n,), jnp.int32),
            pltpu.VMEM((2, ch, 8 * E), jnp.float32),
            pltpu.VMEM((nb, F * E), jnp.float32),
            pltpu.SemaphoreType.DMA,
            pltpu.SemaphoreType.DMA,
        ],
    )
    def gather(tab_hbm, rows_hbm, sub_hbm, dstb_hbm, dstc_hbm, out_hbm,
               rows_v, sub_v, dstb_v, dstc_v, buf, out_v, sem0, sem1):
        wid = lax.axis_index("s") * nc + lax.axis_index("c")
        base = wid * n
        pltpu.sync_copy(rows_hbm.at[pl.ds(base, n)], rows_v)
        pltpu.sync_copy(sub_hbm.at[pl.ds(base, n)], sub_v)
        pltpu.sync_copy(dstb_hbm, dstb_v)
        pltpu.sync_copy(dstc_hbm, dstc_v)
        sems = [sem0, sem1]
        iota = lax.iota(jnp.int32, 16)

        def fire(g, b):
            pltpu.async_copy(
                tab_hbm.at[rows_v.at[pl.ds(g * ch, ch)]], buf.at[b], sems[b])

        def drain(b):
            pltpu.make_async_copy(
                tab_hbm.at[rows_v.at[pl.ds(0, ch)]], buf.at[b], sems[b]).wait()

        def extract(g, b):
            for q in range(ch // 16):
                s0 = g * ch + q * 16
                bat = dstb_v[pl.ds(s0, 16)]
                col0 = dstc_v[pl.ds(s0, 16)]
                sv = sub_v[pl.ds(s0, 16)]
                lrow = q * 16 + iota
                for m in range(E):
                    v = plsc.load_gather(buf.at[b], [lrow, sv + m])
                    plsc.store_scatter(out_v, [bat, col0 + m], v)

        fire(0, 0)

        def pair(p, carry):
            g0 = 2 * p
            fire(g0 + 1, 1)
            drain(0)
            extract(g0, 0)

            @pl.when(g0 + 2 < nch)
            def _():
                fire(g0 + 2, 0)

            drain(1)
            extract(g0 + 1, 1)
            return carry

        lax.fori_loop(0, nch // 2, pair, 0)
        pltpu.sync_copy(out_v, out_hbm.at[pl.ds(wid * nb, nb)])

    return gather


def _dense_body(fe_ref, lw_ref, t2_ref, w1_ref, b1_ref, g1_ref, be1_ref,
                w2_ref, b2_ref, g2_ref, be2_ref, wfc_ref, bfc_ref, out_ref,
                *, lin_dim):
    f32 = jnp.float32
    fe = fe_ref[...]
    lz = jnp.dot(fe, lw_ref[...], preferred_element_type=f32)
    lp = jnp.dot(fe * fe, t2_ref[...], preferred_element_type=f32)
    w1 = w1_ref[...]
    y = (jnp.dot(lz, w1[:lin_dim], preferred_element_type=f32)
         + jnp.dot(lp, w1[lin_dim:], preferred_element_type=f32)
         + b1_ref[...])

    def bn_relu(y, g, b):
        m = jnp.mean(y, axis=0, keepdims=True)
        c = y - m
        v = jnp.mean(c * c, axis=0, keepdims=True)
        return jnp.maximum(g * c * lax.rsqrt(v + 1e-5) + b, 0.0)

    y = bn_relu(y, g1_ref[...], be1_ref[...])
    y = jnp.dot(y, w2_ref[...], preferred_element_type=f32) + b2_ref[...]
    y = bn_relu(y, g2_ref[...], be2_ref[...])
    out_ref[...] = jnp.dot(y, wfc_ref[...], preferred_element_type=f32) + bfc_ref[...]


def _dense(fe, lw, t2, w1, b1, g1, be1, w2, b2, g2, be2, wfc, bfc, lin_dim):
    B = fe.shape[0]
    return pl.pallas_call(
        functools.partial(_dense_body, lin_dim=lin_dim),
        out_shape=jax.ShapeDtypeStruct((B, 1), jnp.float32),
    )(fe, lw, t2, w1, b1, g1, be1, w2, b2, g2, be2, wfc, bfc)


def kernel(feat_index, feat_value, emb_table, linear_weights, theta,
           W1, b1, g1, be1, W2, b2, g2, be2, Wfc, bfc):
    B, F = feat_index.shape
    V, E = emb_table.shape
    lin_dim = linear_weights.shape[0]

    table2 = _make_sc_relayout(V, E)(jnp.transpose(emb_table))

    idx = feat_index.reshape(-1).astype(jnp.int32)
    rows = idx >> 3
    sub = (idx & 7) * E
    n = B * F // 32
    r = jnp.arange(n, dtype=jnp.int32)
    dstb = r // F
    dstc = (r % F) * E
    fe = _make_sc_gather(B, F, E)(table2, rows, sub, dstb, dstc)

    lw = linear_weights.reshape(lin_dim, F * E).T
    t2 = jnp.repeat(theta * theta, E, axis=1).T

    return _dense(fe, lw, t2,
                  W1, b1.reshape(1, -1), g1.reshape(1, -1), be1.reshape(1, -1),
                  W2, b2.reshape(1, -1), g2.reshape(1, -1), be2.reshape(1, -1),
                  Wfc, bfc.reshape(1, -1), lin_dim)


# speedup vs baseline: 2.2448x; 1.0323x over previous
import functools

import jax
import jax.numpy as jnp
from jax import lax
from jax.experimental import pallas as pl
from jax.experimental.pallas import tpu as pltpu
from jax.experimental.pallas import tpu_sc as plsc


def _sc_info():
    info = plsc.get_sparse_core_info()
    return info.num_cores, info.num_cores * info.num_subcores


_CHUNK = 2048


def _make_sc_relayout(V, E):
    nc, nw = _sc_info()
    nfull = V // _CHUNK
    tail = V - nfull * _CHUNK
    per_w = (nfull + nw - 1) // nw
    vpad = ((V + 127) // 128) * 128
    mesh = plsc.VectorSubcoreMesh(core_axis_name="c", subcore_axis_name="s")

    @functools.partial(
        pl.kernel,
        mesh=mesh,
        compiler_params=pltpu.CompilerParams(needs_layout_passes=False),
        out_type=jax.ShapeDtypeStruct((vpad // 8, 8 * E), jnp.float32),
        scratch_types=[
            pltpu.VMEM((E, _CHUNK), jnp.float32),
            pltpu.VMEM((2, _CHUNK // 8, 8 * E), jnp.float32),
            pltpu.SemaphoreType.DMA,
            pltpu.SemaphoreType.DMA,
        ],
    )
    def relayout(tt_hbm, out_hbm, ib, ob, sem0, sem1):
        wid = lax.axis_index("s") * nc + lax.axis_index("c")
        iota = lax.iota(jnp.int32, 16)
        i8 = iota >> 3
        i16 = (iota & 7) * E
        zero = iota * 0
        idx1 = [i16 + j for j in range(E)]
        jv = [zero + j for j in range(E)]
        sems = [sem0, sem1]

        def do_chunk(col0, ncols, b, first):
            col0 = pl.multiple_of(col0, 128)
            pltpu.sync_copy(tt_hbm.at[:, pl.ds(col0, ncols)],
                            ib.at[:, pl.ds(0, ncols)])

            @pl.when(jnp.logical_not(first))
            def _():
                pltpu.make_async_copy(
                    out_hbm.at[pl.ds(0, ncols // 8)],
                    ob.at[b].at[pl.ds(0, ncols // 8)], sems[b]).wait()

            def q_body(q, carry):
                idx0 = 2 * q + i8
                cols = q * 16 + iota
                for j in range(E):
                    v = plsc.load_gather(ib, [jv[j], cols])
                    plsc.store_scatter(ob.at[b], [idx0, idx1[j]], v)
                return carry

            lax.fori_loop(0, ncols // 16, q_body, 0)
            pltpu.async_copy(ob.at[b].at[pl.ds(0, ncols // 8)],
                             out_hbm.at[pl.ds(pl.multiple_of(col0 // 8, 8),
                                              ncols // 8)], sems[b])

        def pair_body(p, carry):
            k0 = 2 * p
            t0 = wid * per_w + k0

            @pl.when(t0 < nfull)
            def _():
                do_chunk(t0 * _CHUNK, _CHUNK, 0, p == 0)

            @pl.when(t0 + 1 < nfull)
            def _():
                do_chunk((t0 + 1) * _CHUNK, _CHUNK, 1, p == 0)

            return carry

        lax.fori_loop(0, per_w // 2, pair_body, 0)

        def fin(b):
            pltpu.make_async_copy(
                out_hbm.at[pl.ds(0, _CHUNK // 8)], ob.at[b], sems[b]).wait()

        @pl.when(wid * per_w < nfull)
        def _():
            fin(0)

        @pl.when(wid * per_w + 1 < nfull)
        def _():
            fin(1)

        if tail:
            t512 = (tail // 512) * 512

            @pl.when(wid == nw - 1)
            def _():
                if t512:
                    col0 = pl.multiple_of(wid * 0 + nfull * _CHUNK, 128)
                    pltpu.sync_copy(tt_hbm.at[:, pl.ds(col0, t512)],
                                    ib.at[:, pl.ds(0, t512)])

                    def q_body(q, carry):
                        idx0 = 2 * q + i8
                        cols = q * 16 + iota
                        for j in range(E):
                            v = plsc.load_gather(ib, [jv[j], cols])
                            plsc.store_scatter(ob.at[0], [idx0, idx1[j]], v)
                        return carry

                    lax.fori_loop(0, t512 // 16, q_body, 0)
                    pltpu.sync_copy(
                        ob.at[0].at[pl.ds(0, t512 // 8)],
                        out_hbm.at[pl.ds(pl.multiple_of(col0 // 8, 8),
                                         t512 // 8)])
                col1 = pl.multiple_of(wid * 0 + nfull * _CHUNK + t512, 128)
                pltpu.sync_copy(tt_hbm.at[:, pl.ds(col1, 128)],
                                ib.at[:, pl.ds(0, 128)])

                def q_body1(q, carry):
                    idx0 = 2 * q + i8
                    cols = q * 16 + iota
                    for j in range(E):
                        v = plsc.load_gather(ib, [jv[j], cols])
                        plsc.store_scatter(ob.at[0], [idx0, idx1[j]], v)
                    return carry

                lax.fori_loop(0, 8, q_body1, 0)
                pltpu.sync_copy(ob.at[0].at[pl.ds(0, 16)],
                                out_hbm.at[pl.ds(pl.multiple_of(
                                    col1 // 8, 8), 16)])

    return relayout


def _make_sc_gather(B, F, E):
    nc, nw = _sc_info()
    n = B * F // nw
    nb = n // F
    ch = 128
    nch = n // ch
    mesh = plsc.VectorSubcoreMesh(core_axis_name="c", subcore_axis_name="s")

    @functools.partial(
        pl.kernel,
        mesh=mesh,
        compiler_params=pltpu.CompilerParams(needs_layout_passes=False),
        out_type=jax.ShapeDtypeStruct((B, F * E), jnp.float32),
        scratch_types=[
            pltpu.VMEM((n,), jnp.int32),
            pltpu.VMEM((n,), jnp.int32),
            pltpu.VMEM((n,), jnp.int32),
            pltpu.VMEM((n,), jnp.int32),
            pltpu.VMEM((2, ch, 8 * E), jnp.float32),
            pltpu.VMEM((nb, F * E), jnp.float32),
            pltpu.SemaphoreType.DMA,
            pltpu.SemaphoreType.DMA,
        ],
    )
    def gather(tab_hbm, rows_hbm, sub_hbm, dstb_hbm, dstc_hbm, out_hbm,
               rows_v, sub_v, dstb_v, dstc_v, buf, out_v, sem0, sem1):
        wid = lax.axis_index("s") * nc + lax.axis_index("c")
        base = wid * n
        pltpu.sync_copy(rows_hbm.at[pl.ds(base, n)], rows_v)
        pltpu.sync_copy(sub_hbm.at[pl.ds(base, n)], sub_v)
        pltpu.sync_copy(dstb_hbm, dstb_v)
        pltpu.sync_copy(dstc_hbm, dstc_v)
        sems = [sem0, sem1]
        iota = lax.iota(jnp.int32, 16)

        def fire(g, b):
            pltpu.async_copy(
                tab_hbm.at[rows_v.at[pl.ds(g * ch, ch)]], buf.at[b], sems[b])

        def drain(b):
            pltpu.make_async_copy(
                tab_hbm.at[rows_v.at[pl.ds(0, ch)]], buf.at[b], sems[b]).wait()

        def extract(g, b):
            for q in range(ch // 16):
                s0 = g * ch + q * 16
                bat = dstb_v[pl.ds(s0, 16)]
                col0 = dstc_v[pl.ds(s0, 16)]
                sv = sub_v[pl.ds(s0, 16)]
                lrow = q * 16 + iota
                for m in range(E):
                    v = plsc.load_gather(buf.at[b], [lrow, sv + m])
                    plsc.store_scatter(out_v, [bat, col0 + m], v)

        fire(0, 0)

        def pair(p, carry):
            g0 = 2 * p
            fire(g0 + 1, 1)
            drain(0)
            extract(g0, 0)

            @pl.when(g0 + 2 < nch)
            def _():
                fire(g0 + 2, 0)

            drain(1)
            extract(g0 + 1, 1)
            return carry

        lax.fori_loop(0, nch // 2, pair, 0)
        pltpu.sync_copy(out_v, out_hbm.at[pl.ds(wid * nb, nb)])

    return gather


def _dense_body(fe_ref, lw_ref, t2_ref, w1_ref, b1_ref, g1_ref, be1_ref,
                w2_ref, b2_ref, g2_ref, be2_ref, wfc_ref, bfc_ref, out_ref,
                *, lin_dim):
    f32 = jnp.float32
    fe = fe_ref[...]
    lz = jnp.dot(fe, lw_ref[...], preferred_element_type=f32)
    lp = jnp.dot(fe * fe, t2_ref[...], preferred_element_type=f32)
    w1 = w1_ref[...]
    y = (jnp.dot(lz, w1[:lin_dim], preferred_element_type=f32)
         + jnp.dot(lp, w1[lin_dim:], preferred_element_type=f32)
         + b1_ref[...])

    def bn_relu(y, g, b):
        m = jnp.mean(y, axis=0, keepdims=True)
        c = y - m
        v = jnp.mean(c * c, axis=0, keepdims=True)
        return jnp.maximum(g * c * lax.rsqrt(v + 1e-5) + b, 0.0)

    y = bn_relu(y, g1_ref[...], be1_ref[...])
    y = jnp.dot(y, w2_ref[...], preferred_element_type=f32) + b2_ref[...]
    y = bn_relu(y, g2_ref[...], be2_ref[...])
    out_ref[...] = jnp.dot(y, wfc_ref[...], preferred_element_type=f32) + bfc_ref[...]


def _dense(fe, lw, t2, w1, b1, g1, be1, w2, b2, g2, be2, wfc, bfc, lin_dim):
    B = fe.shape[0]
    return pl.pallas_call(
        functools.partial(_dense_body, lin_dim=lin_dim),
        out_shape=jax.ShapeDtypeStruct((B, 1), jnp.float32),
    )(fe, lw, t2, w1, b1, g1, be1, w2, b2, g2, be2, wfc, bfc)


def kernel(feat_index, feat_value, emb_table, linear_weights, theta,
           W1, b1, g1, be1, W2, b2, g2, be2, Wfc, bfc):
    B, F = feat_index.shape
    V, E = emb_table.shape
    lin_dim = linear_weights.shape[0]

    table2 = _make_sc_relayout(V, E)(jnp.transpose(emb_table))

    idx = feat_index.reshape(-1).astype(jnp.int32)
    rows = idx >> 3
    sub = (idx & 7) * E
    n = B * F // 32
    r = jnp.arange(n, dtype=jnp.int32)
    dstb = r // F
    dstc = (r % F) * E
    fe = _make_sc_gather(B, F, E)(table2, rows, sub, dstb, dstc)

    lw = linear_weights.reshape(lin_dim, F * E).T
    t2 = jnp.repeat(theta * theta, E, axis=1).T

    return _dense(fe, lw, t2,
                  W1, b1.reshape(1, -1), g1.reshape(1, -1), be1.reshape(1, -1),
                  W2, b2.reshape(1, -1), g2.reshape(1, -1), be2.reshape(1, -1),
                  Wfc, bfc.reshape(1, -1), lin_dim)
